# baseline (device time: 151140 ns/iter reference)
import jax
import jax.numpy as jnp
from jax import lax
from jax.experimental import pallas as pl
from jax.experimental.pallas import tpu as pltpu

N_DEV = 16
B, Sq, Skv, Hq, Dh = 2, 512, 512, 128, 64
H_LOC = Hq // N_DEV
DM = 768
DH_LOC = H_LOC * Dh
ROWS = B * Sq
CH = ROWS // N_DEV


def _allreduce_body(p_ref, out_ref, rs_buf, ag_buf, send_buf,
                    send_sems, rs_sems, ag_sems):
    my = lax.axis_index("i")
    left = lax.rem(my - 1 + N_DEV, N_DEV)
    right = lax.rem(my + 1, N_DEV)

    barrier_sem = pltpu.get_barrier_semaphore()
    for nbr in (left, right):
        pl.semaphore_signal(
            barrier_sem, inc=1,
            device_id=(nbr,), device_id_type=pl.DeviceIdType.MESH,
        )
    pl.semaphore_wait(barrier_sem, 2)

    for s in range(N_DEV - 1):
        c_send = lax.rem(my - s + 2 * N_DEV, N_DEV)
        slot = s % 2
        if s == 0:
            send_buf[slot] = p_ref[pl.ds(c_send * CH, CH), :]
        else:
            send_buf[slot] = rs_buf[s - 1] + p_ref[pl.ds(c_send * CH, CH), :]
        rdma = pltpu.make_async_remote_copy(
            src_ref=send_buf.at[slot],
            dst_ref=rs_buf.at[s],
            send_sem=send_sems.at[slot],
            recv_sem=rs_sems.at[s],
            device_id=(right,),
            device_id_type=pl.DeviceIdType.MESH,
        )
        rdma.start()
        rdma.wait()

    c_own = lax.rem(my + 1, N_DEV)
    reduced = rs_buf[N_DEV - 2] + p_ref[pl.ds(c_own * CH, CH), :]
    out_ref[pl.ds(c_own * CH, CH), :] = reduced
    send_buf[0] = reduced

    for t in range(N_DEV - 1):
        src = send_buf.at[0] if t == 0 else ag_buf.at[t - 1]
        rdma = pltpu.make_async_remote_copy(
            src_ref=src,
            dst_ref=ag_buf.at[t],
            send_sem=send_sems.at[1],
            recv_sem=ag_sems.at[t],
            device_id=(right,),
            device_id_type=pl.DeviceIdType.MESH,
        )
        rdma.start()
        rdma.wait()
        c_recv = lax.rem(my - t + 2 * N_DEV, N_DEV)
        out_ref[pl.ds(c_recv * CH, CH), :] = ag_buf[t]


def _ring_allreduce(partial):
    return pl.pallas_call(
        _allreduce_body,
        out_shape=jax.ShapeDtypeStruct((ROWS, DM), jnp.float32),
        in_specs=[pl.BlockSpec(memory_space=pltpu.VMEM)],
        out_specs=pl.BlockSpec(memory_space=pltpu.VMEM),
        scratch_shapes=[
            pltpu.VMEM((N_DEV - 1, CH, DM), jnp.float32),
            pltpu.VMEM((N_DEV - 1, CH, DM), jnp.float32),
            pltpu.VMEM((2, CH, DM), jnp.float32),
            pltpu.SemaphoreType.DMA((2,)),
            pltpu.SemaphoreType.DMA((N_DEV - 1,)),
            pltpu.SemaphoreType.DMA((N_DEV - 1,)),
        ],
        compiler_params=pltpu.CompilerParams(collective_id=0),
    )(partial)


def kernel(x, Wq, K_ext, V_ext, Wo):
    my = lax.axis_index("i")
    bf = jnp.bfloat16

    Wq_loc = lax.dynamic_slice(Wq, (0, my * DH_LOC), (DM, DH_LOC))
    Wo_loc = lax.dynamic_slice(Wo, (my * DH_LOC, 0), (DH_LOC, DM))

    xf = x.reshape(ROWS, DM).astype(bf)
    Q = jnp.dot(xf, Wq_loc.astype(bf), preferred_element_type=jnp.float32)
    Q = Q.reshape(B, Sq, H_LOC, Dh).astype(bf)
    K = K_ext.astype(bf)
    V = V_ext.astype(bf)

    scores = jnp.einsum(
        "bihd,bjhd->bhij", Q, K, preferred_element_type=jnp.float32
    ) * 0.125
    qi = jnp.arange(Sq)[:, None]
    ki = jnp.arange(Skv)[None, :]
    mask = (jnp.abs(qi - ki) <= 128) | (ki < 32) | (qi < 32)
    scores = jnp.where(mask[None, None], scores, -1e9)
    m = scores.max(axis=-1, keepdims=True)
    w = jnp.exp(scores - m)
    w = w / w.sum(axis=-1, keepdims=True)

    ctx = jnp.einsum(
        "bhij,bjhd->bihd", w.astype(bf), V, preferred_element_type=jnp.float32
    )
    ctx = ctx.reshape(ROWS, DH_LOC).astype(bf)
    partial = jnp.dot(ctx, Wo_loc.astype(bf), preferred_element_type=jnp.float32)

    out = _ring_allreduce(partial)
    return out.reshape(B, Sq, DM)


# device time: 79966 ns/iter; 1.8901x vs baseline; 1.8901x over previous
import jax
import jax.numpy as jnp
from jax import lax
from jax.experimental import pallas as pl
from jax.experimental.pallas import tpu as pltpu

N_DEV = 16
B, Sq, Skv, Hq, Dh = 2, 512, 512, 128, 64
H_LOC = Hq // N_DEV
DM = 768
DH_LOC = H_LOC * Dh
ROWS = B * Sq
CH = ROWS // N_DEV

_MESH = pl.DeviceIdType.MESH


def _partner_and_bit(my, s):
    z = my // 4
    p = my % 4
    x = jnp.where((p == 1) | (p == 2), 1, 0).astype(jnp.int32)
    y = (p // 2).astype(jnp.int32)
    lab = x * 8 + y * 4 + (z % 2) * 2 + (z // 2)
    bit = (lab >> (3 - s)) & 1
    plab = lab ^ (8 >> s)
    px = (plab >> 3) & 1
    py = (plab >> 2) & 1
    pz = (plab & 1) * 2 + ((plab >> 1) & 1)
    pp = py * 2 + (px ^ py)
    return pz * 4 + pp, bit


def _allreduce_body(p_ref, out_ref, acc,
                    rss0, rss1, rss2, rss3, rsr0, rsr1, rsr2, rsr3,
                    ags0, ags1, ags2, ags3, agr0, agr1, agr2, agr3,
                    ss_sems, sr_sems, as_sems, ar_sems):
    my = lax.axis_index("i")
    rs_send = [rss0, rss1, rss2, rss3]
    rs_recv = [rsr0, rsr1, rsr2, rsr3]
    ag_send = [ags0, ags1, ags2, ags3]
    ag_recv = [agr0, agr1, agr2, agr3]

    partners = []
    bits = []
    for s in range(4):
        pt, b = _partner_and_bit(my, s)
        partners.append(pt)
        bits.append(b)

    barrier_sem = pltpu.get_barrier_semaphore()
    for s in range(4):
        pl.semaphore_signal(
            barrier_sem, inc=1, device_id=(partners[s],),
            device_id_type=_MESH,
        )
    pl.semaphore_wait(barrier_sem, 4)

    acc[...] = p_ref[...]

    off = jnp.int32(0)
    for s in range(4):
        sz = 512 >> s
        b = bits[s]
        send_off = pl.multiple_of(off + (1 - b) * sz, CH)
        keep_off = pl.multiple_of(off + b * sz, CH)
        rs_send[s][...] = acc[pl.ds(send_off, sz), :].astype(jnp.bfloat16)
        rdma = pltpu.make_async_remote_copy(
            src_ref=rs_send[s],
            dst_ref=rs_recv[s],
            send_sem=ss_sems.at[s],
            recv_sem=sr_sems.at[s],
            device_id=(partners[s],),
            device_id_type=_MESH,
        )
        rdma.start()
        rdma.wait()
        acc[pl.ds(keep_off, sz), :] = (
            acc[pl.ds(keep_off, sz), :] + rs_recv[s][...].astype(jnp.float32)
        )
        off = keep_off

    off = pl.multiple_of(off, CH)
    out_ref[pl.ds(off, CH), :] = acc[pl.ds(off, CH), :].astype(jnp.bfloat16)

    blk_off = off
    for g in range(4):
        sz = CH << g
        ag_send[g][...] = out_ref[pl.ds(blk_off, sz), :]
        rdma = pltpu.make_async_remote_copy(
            src_ref=ag_send[g],
            dst_ref=ag_recv[g],
            send_sem=as_sems.at[g],
            recv_sem=ar_sems.at[g],
            device_id=(partners[3 - g],),
            device_id_type=_MESH,
        )
        rdma.start()
        rdma.wait()
        p_off = pl.multiple_of(jnp.bitwise_xor(blk_off, CH << g), CH)
        out_ref[pl.ds(p_off, sz), :] = ag_recv[g][...]
        blk_off = pl.multiple_of(jnp.minimum(blk_off, p_off), CH)


def _butterfly_allreduce(partial):
    bf = jnp.bfloat16
    return pl.pallas_call(
        _allreduce_body,
        out_shape=jax.ShapeDtypeStruct((ROWS, DM), bf),
        in_specs=[pl.BlockSpec(memory_space=pltpu.VMEM)],
        out_specs=pl.BlockSpec(memory_space=pltpu.VMEM),
        scratch_shapes=[
            pltpu.VMEM((ROWS, DM), jnp.float32),
            pltpu.VMEM((512, DM), bf), pltpu.VMEM((256, DM), bf),
            pltpu.VMEM((128, DM), bf), pltpu.VMEM((64, DM), bf),
            pltpu.VMEM((512, DM), bf), pltpu.VMEM((256, DM), bf),
            pltpu.VMEM((128, DM), bf), pltpu.VMEM((64, DM), bf),
            pltpu.VMEM((64, DM), bf), pltpu.VMEM((128, DM), bf),
            pltpu.VMEM((256, DM), bf), pltpu.VMEM((512, DM), bf),
            pltpu.VMEM((64, DM), bf), pltpu.VMEM((128, DM), bf),
            pltpu.VMEM((256, DM), bf), pltpu.VMEM((512, DM), bf),
            pltpu.SemaphoreType.DMA((4,)),
            pltpu.SemaphoreType.DMA((4,)),
            pltpu.SemaphoreType.DMA((4,)),
            pltpu.SemaphoreType.DMA((4,)),
        ],
        compiler_params=pltpu.CompilerParams(collective_id=0),
    )(partial)


def kernel(x, Wq, K_ext, V_ext, Wo):
    my = lax.axis_index("i")
    bf = jnp.bfloat16

    Wq_loc = lax.dynamic_slice(Wq, (0, my * DH_LOC), (DM, DH_LOC))
    Wo_loc = lax.dynamic_slice(Wo, (my * DH_LOC, 0), (DH_LOC, DM))

    xf = x.reshape(ROWS, DM).astype(bf)
    Q = jnp.dot(xf, Wq_loc.astype(bf), preferred_element_type=jnp.float32)
    Q = Q.reshape(B, Sq, H_LOC, Dh).astype(bf)
    K = K_ext.astype(bf)
    V = V_ext.astype(bf)

    scores = jnp.einsum(
        "bihd,bjhd->bhij", Q, K, preferred_element_type=jnp.float32
    ) * 0.125
    qi = jnp.arange(Sq)[:, None]
    ki = jnp.arange(Skv)[None, :]
    mask = (jnp.abs(qi - ki) <= 128) | (ki < 32) | (qi < 32)
    scores = jnp.where(mask[None, None], scores, -1e9)
    m = scores.max(axis=-1, keepdims=True)
    w = jnp.exp(scores - m)
    w = w / w.sum(axis=-1, keepdims=True)

    ctx = jnp.einsum(
        "bhij,bjhd->bihd", w.astype(bf), V, preferred_element_type=jnp.float32
    )
    ctx = ctx.reshape(ROWS, DH_LOC).astype(bf)
    partial = jnp.dot(ctx, Wo_loc.astype(bf), preferred_element_type=jnp.float32)

    out = _butterfly_allreduce(partial)
    return out.reshape(B, Sq, DM)


# device time: 67495 ns/iter; 2.2393x vs baseline; 1.1848x over previous
import jax
import jax.numpy as jnp
from jax import lax
from jax.experimental import pallas as pl
from jax.experimental.pallas import tpu as pltpu

N_DEV = 16
B, Sq, Skv, Hq, Dh = 2, 512, 512, 128, 64
H_LOC = Hq // N_DEV
DM = 768
DH_LOC = H_LOC * Dh
ROWS = B * Sq
CH = ROWS // N_DEV

_MESH = pl.DeviceIdType.MESH


def _partner_and_bit(my, s):
    z = my // 4
    p = my % 4
    x = jnp.where((p == 1) | (p == 2), 1, 0).astype(jnp.int32)
    y = (p // 2).astype(jnp.int32)
    lab = x * 8 + y * 4 + (z % 2) * 2 + (z // 2)
    bit = (lab >> (3 - s)) & 1
    plab = lab ^ (8 >> s)
    px = (plab >> 3) & 1
    py = (plab >> 2) & 1
    pz = (plab & 1) * 2 + ((plab >> 1) & 1)
    pp = py * 2 + (px ^ py)
    return pz * 4 + pp, bit


def _fused_body(x_ref, wq_ref, kt_ref, vt_ref, wo_ref, out_ref,
                q_buf, ctx_buf, acc,
                rss0, rss1, rss2, rss3, rsr0, rsr1, rsr2, rsr3,
                ags0, ags1, ags2, ags3, agr0, agr1, agr2, agr3,
                ss_sems, sr_sems, as_sems, ar_sems):
    my = lax.axis_index("i")
    bf = jnp.bfloat16
    rs_send = [rss0, rss1, rss2, rss3]
    rs_recv = [rsr0, rsr1, rsr2, rsr3]
    ag_send = [ags0, ags1, ags2, ags3]
    ag_recv = [agr0, agr1, agr2, agr3]

    partners = []
    bits = []
    for s in range(4):
        pt, b = _partner_and_bit(my, s)
        partners.append(pt)
        bits.append(b)

    barrier_sem = pltpu.get_barrier_semaphore()
    for s in range(4):
        pl.semaphore_signal(
            barrier_sem, inc=1, device_id=(partners[s],),
            device_id_type=_MESH,
        )

    q_buf[...] = (
        jnp.dot(x_ref[...], wq_ref[...], preferred_element_type=jnp.float32)
        * 0.125
    ).astype(bf)

    qi = lax.broadcasted_iota(jnp.int32, (Sq, Skv), 0)
    ki = lax.broadcasted_iota(jnp.int32, (Sq, Skv), 1)
    mask = (jnp.abs(qi - ki) <= 128) | (ki < 32) | (qi < 32)

    for bh in range(B * H_LOC):
        b, h = divmod(bh, H_LOC)
        q = q_buf[b * Sq:(b + 1) * Sq, h * Dh:(h + 1) * Dh]
        k = kt_ref[bh]
        v = vt_ref[bh]
        scores = lax.dot_general(
            q, k, (((1,), (1,)), ((), ())),
            preferred_element_type=jnp.float32,
        )
        scores = jnp.where(mask, scores, -1e9)
        m = jnp.max(scores, axis=1, keepdims=True)
        e = jnp.exp(scores - m)
        w = (e / jnp.sum(e, axis=1, keepdims=True)).astype(bf)
        ctx = lax.dot_general(
            w, v, (((1,), (0,)), ((), ())),
            preferred_element_type=jnp.float32,
        )
        ctx_buf[b * Sq:(b + 1) * Sq, h * Dh:(h + 1) * Dh] = ctx.astype(bf)

    acc[...] = jnp.dot(
        ctx_buf[...], wo_ref[...], preferred_element_type=jnp.float32
    )

    pl.semaphore_wait(barrier_sem, 4)

    off = jnp.int32(0)
    for s in range(4):
        sz = 512 >> s
        b = bits[s]
        send_off = pl.multiple_of(off + (1 - b) * sz, CH)
        keep_off = pl.multiple_of(off + b * sz, CH)
        rs_send[s][...] = acc[pl.ds(send_off, sz), :].astype(bf)
        rdma = pltpu.make_async_remote_copy(
            src_ref=rs_send[s],
            dst_ref=rs_recv[s],
            send_sem=ss_sems.at[s],
            recv_sem=sr_sems.at[s],
            device_id=(partners[s],),
            device_id_type=_MESH,
        )
        rdma.start()
        rdma.wait()
        acc[pl.ds(keep_off, sz), :] = (
            acc[pl.ds(keep_off, sz), :] + rs_recv[s][...].astype(jnp.float32)
        )
        off = keep_off

    off = pl.multiple_of(off, CH)
    out_ref[pl.ds(off, CH), :] = acc[pl.ds(off, CH), :].astype(bf)

    blk_off = off
    for g in range(4):
        sz = CH << g
        ag_send[g][...] = out_ref[pl.ds(blk_off, sz), :]
        rdma = pltpu.make_async_remote_copy(
            src_ref=ag_send[g],
            dst_ref=ag_recv[g],
            send_sem=as_sems.at[g],
            recv_sem=ar_sems.at[g],
            device_id=(partners[3 - g],),
            device_id_type=_MESH,
        )
        rdma.start()
        rdma.wait()
        p_off = pl.multiple_of(jnp.bitwise_xor(blk_off, CH << g), CH)
        out_ref[pl.ds(p_off, sz), :] = ag_recv[g][...]
        blk_off = pl.multiple_of(jnp.minimum(blk_off, p_off), CH)


def kernel(x, Wq, K_ext, V_ext, Wo):
    my = lax.axis_index("i")
    bf = jnp.bfloat16

    Wq_loc = lax.dynamic_slice(Wq, (0, my * DH_LOC), (DM, DH_LOC)).astype(bf)
    Wo_loc = lax.dynamic_slice(Wo, (my * DH_LOC, 0), (DH_LOC, DM)).astype(bf)
    x_bf = x.reshape(ROWS, DM).astype(bf)
    K_t = K_ext.transpose(0, 2, 1, 3).reshape(B * H_LOC, Skv, Dh).astype(bf)
    V_t = V_ext.transpose(0, 2, 1, 3).reshape(B * H_LOC, Skv, Dh).astype(bf)

    vmem = pl.BlockSpec(memory_space=pltpu.VMEM)
    out = pl.pallas_call(
        _fused_body,
        out_shape=jax.ShapeDtypeStruct((ROWS, DM), bf),
        in_specs=[vmem] * 5,
        out_specs=vmem,
        scratch_shapes=[
            pltpu.VMEM((ROWS, DH_LOC), bf),
            pltpu.VMEM((ROWS, DH_LOC), bf),
            pltpu.VMEM((ROWS, DM), jnp.float32),
            pltpu.VMEM((512, DM), bf), pltpu.VMEM((256, DM), bf),
            pltpu.VMEM((128, DM), bf), pltpu.VMEM((64, DM), bf),
            pltpu.VMEM((512, DM), bf), pltpu.VMEM((256, DM), bf),
            pltpu.VMEM((128, DM), bf), pltpu.VMEM((64, DM), bf),
            pltpu.VMEM((64, DM), bf), pltpu.VMEM((128, DM), bf),
            pltpu.VMEM((256, DM), bf), pltpu.VMEM((512, DM), bf),
            pltpu.VMEM((64, DM), bf), pltpu.VMEM((128, DM), bf),
            pltpu.VMEM((256, DM), bf), pltpu.VMEM((512, DM), bf),
            pltpu.SemaphoreType.DMA((4,)),
            pltpu.SemaphoreType.DMA((4,)),
            pltpu.SemaphoreType.DMA((4,)),
            pltpu.SemaphoreType.DMA((4,)),
        ],
        compiler_params=pltpu.CompilerParams(collective_id=0),
    )(x_bf, Wq_loc, K_t, V_t, Wo_loc)
    return out.reshape(B, Sq, DM)


# device time: 64133 ns/iter; 2.3567x vs baseline; 1.0524x over previous
import jax
import jax.numpy as jnp
from jax import lax
from jax.experimental import pallas as pl
from jax.experimental.pallas import tpu as pltpu

N_DEV = 16
B, Sq, Skv, Hq, Dh = 2, 512, 512, 128, 64
H_LOC = Hq // N_DEV
DM = 768
DH_LOC = H_LOC * Dh
ROWS = B * Sq
CH = ROWS // N_DEV

_MESH = pl.DeviceIdType.MESH


def _partner_and_bit(my, s):
    z = my // 4
    p = my % 4
    x = jnp.where((p == 1) | (p == 2), 1, 0).astype(jnp.int32)
    y = (p // 2).astype(jnp.int32)
    lab = x * 8 + y * 4 + (z % 2) * 2 + (z // 2)
    bit = (lab >> (3 - s)) & 1
    plab = lab ^ (8 >> s)
    px = (plab >> 3) & 1
    py = (plab >> 2) & 1
    pz = (plab & 1) * 2 + ((plab >> 1) & 1)
    pp = py * 2 + (px ^ py)
    return pz * 4 + pp, bit


def _fused_body(x_ref, wq_ref, kt_ref, vt_ref, wo_ref, out_ref,
                q_buf, ctx_buf, acc,
                rss0, rss1, rss2, rss3, rsr0, rsr1, rsr2, rsr3,
                ss_sems, sr_sems, as_sems, ar_sems):
    my = lax.axis_index("i")
    bf = jnp.bfloat16
    rs_send = [rss0, rss1, rss2, rss3]
    rs_recv = [rsr0, rsr1, rsr2, rsr3]

    partners = []
    bits = []
    for s in range(4):
        pt, b = _partner_and_bit(my, s)
        partners.append(pt)
        bits.append(b)

    barrier_sem = pltpu.get_barrier_semaphore()
    for s in range(4):
        pl.semaphore_signal(
            barrier_sem, inc=1, device_id=(partners[s],),
            device_id_type=_MESH,
        )

    q_buf[...] = (
        jnp.dot(x_ref[...], wq_ref[...], preferred_element_type=jnp.float32)
        * 0.125
    ).astype(bf)

    qi = lax.broadcasted_iota(jnp.int32, (Sq, Skv), 0)
    ki = lax.broadcasted_iota(jnp.int32, (Sq, Skv), 1)
    mask = (jnp.abs(qi - ki) <= 128) | (ki < 32) | (qi < 32)

    def attn_batch(bs):
        for h in range(H_LOC):
            bh = bs * H_LOC + h
            q = q_buf[bs * Sq:(bs + 1) * Sq, h * Dh:(h + 1) * Dh]
            k = kt_ref[bh]
            v = vt_ref[bh]
            scores = lax.dot_general(
                q, k, (((1,), (1,)), ((), ())),
                preferred_element_type=jnp.float32,
            )
            scores = jnp.where(mask, scores, -1e9)
            m = jnp.max(scores, axis=1, keepdims=True)
            e = jnp.exp(scores - m)
            w = (e / jnp.sum(e, axis=1, keepdims=True)).astype(bf)
            ctx = lax.dot_general(
                w, v, (((1,), (0,)), ((), ())),
                preferred_element_type=jnp.float32,
            )
            ctx_buf[bs * Sq:(bs + 1) * Sq, h * Dh:(h + 1) * Dh] = ctx.astype(bf)

    def partial_batch(bs):
        return jnp.dot(
            ctx_buf[bs * Sq:(bs + 1) * Sq, :], wo_ref[...],
            preferred_element_type=jnp.float32,
        )

    b0 = bits[0]

    @pl.when(b0 == 0)
    def _():
        attn_batch(1)
        rss0[...] = partial_batch(1).astype(bf)

    @pl.when(b0 == 1)
    def _():
        attn_batch(0)
        rss0[...] = partial_batch(0).astype(bf)

    pl.semaphore_wait(barrier_sem, 4)
    rdma0 = pltpu.make_async_remote_copy(
        src_ref=rss0,
        dst_ref=rsr0,
        send_sem=ss_sems.at[0],
        recv_sem=sr_sems.at[0],
        device_id=(partners[0],),
        device_id_type=_MESH,
    )
    rdma0.start()

    @pl.when(b0 == 0)
    def _():
        attn_batch(0)
        acc[0:Sq, :] = partial_batch(0)

    @pl.when(b0 == 1)
    def _():
        attn_batch(1)
        acc[Sq:2 * Sq, :] = partial_batch(1)

    rdma0.wait()
    off = pl.multiple_of(b0 * Sq, CH)
    acc[pl.ds(off, Sq), :] = (
        acc[pl.ds(off, Sq), :] + rsr0[...].astype(jnp.float32)
    )

    for s in range(1, 4):
        sz = 512 >> s
        b = bits[s]
        send_off = pl.multiple_of(off + (1 - b) * sz, CH)
        keep_off = pl.multiple_of(off + b * sz, CH)
        rs_send[s][...] = acc[pl.ds(send_off, sz), :].astype(bf)
        rdma = pltpu.make_async_remote_copy(
            src_ref=rs_send[s],
            dst_ref=rs_recv[s],
            send_sem=ss_sems.at[s],
            recv_sem=sr_sems.at[s],
            device_id=(partners[s],),
            device_id_type=_MESH,
        )
        rdma.start()
        rdma.wait()
        acc[pl.ds(keep_off, sz), :] = (
            acc[pl.ds(keep_off, sz), :] + rs_recv[s][...].astype(jnp.float32)
        )
        off = keep_off

    off = pl.multiple_of(off, CH)
    out_ref[pl.ds(off, CH), :] = acc[pl.ds(off, CH), :].astype(bf)

    blk_off = off
    for g in range(4):
        sz = CH << g
        rdma = pltpu.make_async_remote_copy(
            src_ref=out_ref.at[pl.ds(blk_off, sz), :],
            dst_ref=out_ref.at[pl.ds(blk_off, sz), :],
            send_sem=as_sems.at[g],
            recv_sem=ar_sems.at[g],
            device_id=(partners[3 - g],),
            device_id_type=_MESH,
        )
        rdma.start()
        rdma.wait()
        p_off = pl.multiple_of(jnp.bitwise_xor(blk_off, CH << g), CH)
        blk_off = pl.multiple_of(jnp.minimum(blk_off, p_off), CH)


def kernel(x, Wq, K_ext, V_ext, Wo):
    my = lax.axis_index("i")
    bf = jnp.bfloat16

    Wq_loc = lax.dynamic_slice(Wq, (0, my * DH_LOC), (DM, DH_LOC)).astype(bf)
    Wo_loc = lax.dynamic_slice(Wo, (my * DH_LOC, 0), (DH_LOC, DM)).astype(bf)
    x_bf = x.reshape(ROWS, DM).astype(bf)
    K_t = K_ext.transpose(0, 2, 1, 3).reshape(B * H_LOC, Skv, Dh).astype(bf)
    V_t = V_ext.transpose(0, 2, 1, 3).reshape(B * H_LOC, Skv, Dh).astype(bf)

    vmem = pl.BlockSpec(memory_space=pltpu.VMEM)
    out = pl.pallas_call(
        _fused_body,
        out_shape=jax.ShapeDtypeStruct((ROWS, DM), bf),
        in_specs=[vmem] * 5,
        out_specs=vmem,
        scratch_shapes=[
            pltpu.VMEM((ROWS, DH_LOC), bf),
            pltpu.VMEM((ROWS, DH_LOC), bf),
            pltpu.VMEM((ROWS, DM), jnp.float32),
            pltpu.VMEM((512, DM), bf), pltpu.VMEM((256, DM), bf),
            pltpu.VMEM((128, DM), bf), pltpu.VMEM((64, DM), bf),
            pltpu.VMEM((512, DM), bf), pltpu.VMEM((256, DM), bf),
            pltpu.VMEM((128, DM), bf), pltpu.VMEM((64, DM), bf),
            pltpu.SemaphoreType.DMA((4,)),
            pltpu.SemaphoreType.DMA((4,)),
            pltpu.SemaphoreType.DMA((4,)),
            pltpu.SemaphoreType.DMA((4,)),
        ],
        compiler_params=pltpu.CompilerParams(collective_id=0),
    )(x_bf, Wq_loc, K_t, V_t, Wo_loc)
    return out.reshape(B, Sq, DM)


# device time: 54321 ns/iter; 2.7823x vs baseline; 1.1806x over previous
import jax
import jax.numpy as jnp
from jax import lax
from jax.experimental import pallas as pl
from jax.experimental.pallas import tpu as pltpu

N_DEV = 16
B, Sq, Skv, Hq, Dh = 2, 512, 512, 128, 64
H_LOC = Hq // N_DEV
DM = 768
DH_LOC = H_LOC * Dh
ROWS = B * Sq
QR = ROWS // 4
CH = ROWS // N_DEV

_MESH = pl.DeviceIdType.MESH
_P_XOR = [1, 3, 2]
_Q_XOR = [2, 1, 3]


def _fused_body(x_ref, wq_ref, kt_ref, vt_ref, wo_ref, out_ref,
                q_buf, ctx_buf, acc, ps1, pr1, ps2, pr2,
                s1s, s1r, s2s, s2r, s3s, s3r, s4s, s4r):
    bf = jnp.bfloat16
    my = lax.axis_index("i")
    z = my // 4
    p = my % 4
    xb = jnp.where((p == 1) | (p == 2), 1, 0).astype(jnp.int32)
    yb = (p // 2).astype(jnp.int32)
    q = xb * 2 + yb
    base = pl.multiple_of(q * QR, CH)

    plane_peer = [z * 4 + jnp.bitwise_xor(p, c) for c in _P_XOR]
    z_peer = [jnp.bitwise_xor(z, r + 1) * 4 + p for r in range(3)]

    barrier_sem = pltpu.get_barrier_semaphore()
    for nbr in plane_peer + z_peer:
        pl.semaphore_signal(
            barrier_sem, inc=1, device_id=(nbr,), device_id_type=_MESH,
        )

    q_buf[...] = (
        jnp.dot(x_ref[...], wq_ref[...], preferred_element_type=jnp.float32)
        * 0.125
    ).astype(bf)

    qi = lax.broadcasted_iota(jnp.int32, (Sq, Skv), 0)
    ki = lax.broadcasted_iota(jnp.int32, (Sq, Skv), 1)
    mask = (jnp.abs(qi - ki) <= 128) | (ki < 32) | (qi < 32)

    def attn_batch(bs):
        for h in range(H_LOC):
            bh = bs * H_LOC + h
            qh = q_buf[bs * Sq:(bs + 1) * Sq, h * Dh:(h + 1) * Dh]
            k = kt_ref[bh]
            v = vt_ref[bh]
            scores = lax.dot_general(
                qh, k, (((1,), (1,)), ((), ())),
                preferred_element_type=jnp.float32,
            )
            scores = jnp.where(mask, scores, -1e9)
            m = jnp.max(scores, axis=1, keepdims=True)
            e = jnp.exp(scores - m)
            w = (e / jnp.sum(e, axis=1, keepdims=True)).astype(bf)
            ctx = lax.dot_general(
                w, v, (((1,), (0,)), ((), ())),
                preferred_element_type=jnp.float32,
            )
            ctx_buf[bs * Sq:(bs + 1) * Sq, h * Dh:(h + 1) * Dh] = ctx.astype(bf)

    def partial_batch(bs):
        return jnp.dot(
            ctx_buf[bs * Sq:(bs + 1) * Sq, :], wo_ref[...],
            preferred_element_type=jnp.float32,
        )

    def s1_desc(r):
        return pltpu.make_async_remote_copy(
            src_ref=ps1.at[r],
            dst_ref=pr1.at[r],
            send_sem=s1s.at[r],
            recv_sem=s1r.at[r],
            device_id=(plane_peer[r],),
            device_id_type=_MESH,
        )

    def stage_s1(r):
        qq = pl.multiple_of(jnp.bitwise_xor(q, _Q_XOR[r]) * QR, CH)
        ps1[r, :, :] = acc[pl.ds(qq, QR), :].astype(bf)

    @pl.when(xb == 0)
    def _():
        attn_batch(1)
        acc[Sq:2 * Sq, :] = partial_batch(1)

    @pl.when(xb == 1)
    def _():
        attn_batch(0)
        acc[0:Sq, :] = partial_batch(0)

    stage_s1(0)
    stage_s1(2)
    pl.semaphore_wait(barrier_sem, 6)
    d1 = [s1_desc(r) for r in range(3)]
    d1[0].start()
    d1[2].start()

    @pl.when(xb == 0)
    def _():
        attn_batch(0)
        acc[0:Sq, :] = partial_batch(0)

    @pl.when(xb == 1)
    def _():
        attn_batch(1)
        acc[Sq:2 * Sq, :] = partial_batch(1)

    stage_s1(1)
    d1[1].start()

    for r in range(3):
        d1[r].wait()
        acc[pl.ds(base, QR), :] = (
            acc[pl.ds(base, QR), :] + pr1[r].astype(jnp.float32)
        )

    d2 = []
    for r in range(3):
        zz = jnp.bitwise_xor(z, r + 1)
        off2 = pl.multiple_of(base + zz * CH, CH)
        ps2[r, :, :] = acc[pl.ds(off2, CH), :].astype(bf)
        d = pltpu.make_async_remote_copy(
            src_ref=ps2.at[r],
            dst_ref=pr2.at[r],
            send_sem=s2s.at[r],
            recv_sem=s2r.at[r],
            device_id=(z_peer[r],),
            device_id_type=_MESH,
        )
        d.start()
        d2.append(d)
    fin = pl.multiple_of(base + z * CH, CH)
    for r in range(3):
        d2[r].wait()
        acc[pl.ds(fin, CH), :] = (
            acc[pl.ds(fin, CH), :] + pr2[r].astype(jnp.float32)
        )
    out_ref[pl.ds(fin, CH), :] = acc[pl.ds(fin, CH), :].astype(bf)

    d3 = []
    for r in range(3):
        d = pltpu.make_async_remote_copy(
            src_ref=out_ref.at[pl.ds(fin, CH), :],
            dst_ref=out_ref.at[pl.ds(fin, CH), :],
            send_sem=s3s.at[r],
            recv_sem=s3r.at[r],
            device_id=(z_peer[r],),
            device_id_type=_MESH,
        )
        d.start()
        d3.append(d)
    for r in range(3):
        d3[r].wait()

    d4 = []
    for r in range(3):
        d = pltpu.make_async_remote_copy(
            src_ref=out_ref.at[pl.ds(base, QR), :],
            dst_ref=out_ref.at[pl.ds(base, QR), :],
            send_sem=s4s.at[r],
            recv_sem=s4r.at[r],
            device_id=(plane_peer[r],),
            device_id_type=_MESH,
        )
        d.start()
        d4.append(d)
    for r in range(3):
        d4[r].wait()


def kernel(x, Wq, K_ext, V_ext, Wo):
    my = lax.axis_index("i")
    bf = jnp.bfloat16

    Wq_loc = lax.dynamic_slice(Wq, (0, my * DH_LOC), (DM, DH_LOC)).astype(bf)
    Wo_loc = lax.dynamic_slice(Wo, (my * DH_LOC, 0), (DH_LOC, DM)).astype(bf)
    x_bf = x.reshape(ROWS, DM).astype(bf)
    K_t = K_ext.transpose(0, 2, 1, 3).reshape(B * H_LOC, Skv, Dh).astype(bf)
    V_t = V_ext.transpose(0, 2, 1, 3).reshape(B * H_LOC, Skv, Dh).astype(bf)

    vmem = pl.BlockSpec(memory_space=pltpu.VMEM)
    out = pl.pallas_call(
        _fused_body,
        out_shape=jax.ShapeDtypeStruct((ROWS, DM), bf),
        in_specs=[vmem] * 5,
        out_specs=vmem,
        scratch_shapes=[
            pltpu.VMEM((ROWS, DH_LOC), bf),
            pltpu.VMEM((ROWS, DH_LOC), bf),
            pltpu.VMEM((ROWS, DM), jnp.float32),
            pltpu.VMEM((3, QR, DM), bf),
            pltpu.VMEM((3, QR, DM), bf),
            pltpu.VMEM((3, CH, DM), bf),
            pltpu.VMEM((3, CH, DM), bf),
            pltpu.SemaphoreType.DMA((3,)),
            pltpu.SemaphoreType.DMA((3,)),
            pltpu.SemaphoreType.DMA((3,)),
            pltpu.SemaphoreType.DMA((3,)),
            pltpu.SemaphoreType.DMA((3,)),
            pltpu.SemaphoreType.DMA((3,)),
            pltpu.SemaphoreType.DMA((3,)),
            pltpu.SemaphoreType.DMA((3,)),
        ],
        compiler_params=pltpu.CompilerParams(collective_id=0),
    )(x_bf, Wq_loc, K_t, V_t, Wo_loc)
    return out.reshape(B, Sq, DM)
